# baseline (speedup 1.0000x reference)
import functools

import jax
import jax.numpy as jnp
from jax import lax
from jax.experimental import pallas as pl
from jax.experimental.pallas import tpu as pltpu
from jax.experimental.pallas import tpu_sc as plsc

S = 512
H = 256
D = 128
MAXG = 2

_F32 = jnp.float32


def _sc_gather(token_table, gaz_table, tok_idx, fw_idx, bw_idx):
    mesh = plsc.VectorSubcoreMesh(core_axis_name="c", subcore_axis_name="s")

    @functools.partial(
        pl.kernel,
        mesh=mesh,
        out_type=[
            jax.ShapeDtypeStruct((S, D), _F32),
            jax.ShapeDtypeStruct((S * 8, D), _F32),
            jax.ShapeDtypeStruct((S * 8, D), _F32),
        ],
        scratch_types=[
            pltpu.VMEM((16,), jnp.int32),
            pltpu.VMEM((16, D), _F32),
            pltpu.VMEM((128,), jnp.int32),
            pltpu.VMEM((128, D), _F32),
            pltpu.VMEM((128,), jnp.int32),
            pltpu.VMEM((128, D), _F32),
            pltpu.SemaphoreType.DMA,
        ],
    )
    def gk(tok_tab, gaz_tab, t_idx, f_idx, b_idx, emb_o, fwg_o, bwg_o,
           ti_v, tr_v, fi_v, fr_v, bi_v, br_v, sem):
        wid = lax.axis_index("s") * 2 + lax.axis_index("c")
        pltpu.sync_copy(t_idx.at[pl.ds(wid * 16, 16)], ti_v)
        pltpu.async_copy(tok_tab.at[ti_v], tr_v, sem).wait()
        pltpu.sync_copy(tr_v, emb_o.at[pl.ds(wid * 16, 16)])
        pltpu.sync_copy(f_idx.at[pl.ds(wid * 128, 128)], fi_v)
        pltpu.async_copy(gaz_tab.at[fi_v], fr_v, sem).wait()
        pltpu.sync_copy(fr_v, fwg_o.at[pl.ds(wid * 128, 128)])
        pltpu.sync_copy(b_idx.at[pl.ds(wid * 128, 128)], bi_v)
        pltpu.async_copy(gaz_tab.at[bi_v], br_v, sem).wait()
        pltpu.sync_copy(br_v, bwg_o.at[pl.ds(wid * 128, 128)])

    return gk(token_table, gaz_table, tok_idx, fw_idx, bw_idx)


def _dot(a, b):
    return jnp.dot(a, b, preferred_element_type=_F32)


def _dotb(a, b):
    return jnp.dot(a.astype(jnp.bfloat16), b, preferred_element_type=_F32)


def _shift3(A1, A2, new6):
    z4 = jnp.zeros((4, H), _F32)
    A0n = jnp.concatenate([A1[0:4], new6[0:2]], axis=0)
    A1n = jnp.concatenate([A2[0:2], new6[2:4], A2[4:6]], axis=0)
    A2n = jnp.concatenate([new6[4:6], z4], axis=0)
    return A0n, A1n, A2n


def _gates(g4):
    sg = jax.nn.sigmoid(g4[:, :3 * H])
    return sg[:, :H], sg[:, H:2 * H], sg[:, 2 * H:], jnp.tanh(g4[:, 3 * H:])


def _cnew(c, B0, M0, aw, i_g, f_g, g_g):
    ew = M0 * jnp.exp(jax.nn.sigmoid(aw))
    e0 = jnp.exp(i_g)
    s_e = jnp.sum(ew, axis=0, keepdims=True)
    s_ec = jnp.sum(ew * B0, axis=0, keepdims=True)
    anym = jnp.max(M0, axis=0, keepdims=True)
    c_multi = (e0 * g_g + s_ec) / (e0 + s_e)
    c_plain = f_g * c + i_g * g_g
    return jnp.where(anym > 0.5, c_multi, c_plain)


def _wordcells(wg, c_new):
    sg = jax.nn.sigmoid(wg[:, :2 * H])
    iw, fw_, gw = sg[:, :H], sg[:, H:], jnp.tanh(wg[:, 2 * H:])
    return fw_ * c_new + iw * gw


def _tc_body(emb, gefw, gebw, vbf_ref, vbb_ref,
             fwWh, fwWwx, fwWwh, fwbwb, fwWlc,
             bwWh, bwWwx, bwWwh, bwbwb, bwWlc,
             fwWx, fwb, fwWlx, fwbl, bwWx, bwb, bwWlx, bwbl,
             hs_ref, xwf, xlf, xwb, xlb, gxf, gxb):
    for ci in range(8):
        r0 = ci * 64
        e = emb[r0:r0 + 64, :]
        xwf[r0:r0 + 64, :] = _dot(e, fwWx[:, :]) + fwb[:, :]
        xlf[r0:r0 + 64, :] = _dot(e, fwWlx[:, :]) + fwbl[:, :]
        xwb[r0:r0 + 64, :] = _dot(e, bwWx[:, :]) + bwb[:, :]
        xlb[r0:r0 + 64, :] = _dot(e, bwWlx[:, :]) + bwbl[:, :]
    for ci in range(16):
        r0 = ci * 256
        gxf[r0:r0 + 256, :] = _dotb(gefw[r0:r0 + 256, :], fwWwx[:, :]) + fwbwb[:, :]
        gxb[r0:r0 + 256, :] = _dotb(gebw[r0:r0 + 256, :], bwWwx[:, :]) + bwbwb[:, :]

    z1 = jnp.zeros((1, H), _F32)
    z6 = jnp.zeros((6, H), _F32)
    init = (z1, z1, z6, z6, z6, z6, z6, z6, z6, z6,
            xwf[0:1, :], jnp.broadcast_to(xlf[0:1, :], (6, H)),
            z1, z1, z6, z6, z6, z6, z6, z6, z6, z6,
            xwb[S - 1:S, :], jnp.broadcast_to(xlb[S - 1:S, :], (6, H)))

    def body(t, carry):
        (hf, cf, B0f, B1f, B2f, M0f, M1f, M2f, W1f, W2f, g4f, awf,
         hb, cb, B0b, B1b, B2b, M0b, M1b, M2b, W1b, W2b, g4b, awb) = carry
        p = S - 1 - t
        tn = jnp.minimum(t + 1, S - 1)
        pn = jnp.maximum(p - 1, 0)
        if_, ff, of, gf = _gates(g4f)
        ib_, fb, ob, gb = _gates(g4b)
        cf_n = _cnew(cf, B0f, M0f, awf, if_, ff, gf)
        cb_n = _cnew(cb, B0b, M0b, awb, ib_, fb, gb)
        hf_n = of * jnp.tanh(cf_n)
        hb_n = ob * jnp.tanh(cb_n)
        hs_ref[pl.ds(t, 1), 0:H] = hf_n
        hs_ref[pl.ds(p, 1), H:2 * H] = hb_n
        gx_f = gxf[pl.ds(8 * t, 8), :][0:2]
        gx_b = gxb[pl.ds(8 * p, 8), :][0:6]
        vb_f = vbf_ref[pl.ds(8 * t, 8), :][0:6]
        vb_b = vbb_ref[pl.ds(8 * p, 8), :][0:6]
        wgf = gx_f + _dotb(hf_n, fwWwh[:, :])
        wgb = gx_b + _dotb(hb_n, bwWwh[:, :])
        cwf = _wordcells(wgf, cf_n)
        cwb = _wordcells(wgb, cb_n)
        cwWf = _dotb(cwf, fwWlc[:, :])
        cwWb = _dotb(cwb, bwWlc[:, :])
        cw6f = jnp.concatenate([cwf, cwf, cwf], axis=0)
        cwW6f = jnp.concatenate([cwWf, cwWf, cwWf], axis=0)
        B0fn, B1fn, B2fn = _shift3(B1f, B2f, cw6f)
        B0bn, B1bn, B2bn = _shift3(B1b, B2b, cwb)
        M0fn, M1fn, M2fn = _shift3(M1f, M2f, vb_f)
        M0bn, M1bn, M2bn = _shift3(M1b, M2b, vb_b)
        W0fn, W1fn, W2fn = _shift3(W1f, W2f, cwW6f)
        W0bn, W1bn, W2bn = _shift3(W1b, W2b, cwWb)
        g4f_n = xwf[pl.ds(tn, 1), :] + _dotb(hf_n, fwWh[:, :])
        g4b_n = xwb[pl.ds(pn, 1), :] + _dotb(hb_n, bwWh[:, :])
        awf_n = xlf[pl.ds(tn, 1), :] + W0fn
        awb_n = xlb[pl.ds(pn, 1), :] + W0bn
        return (hf_n, cf_n, B0fn, B1fn, B2fn, M0fn, M1fn, M2fn,
                W1fn, W2fn, g4f_n, awf_n,
                hb_n, cb_n, B0bn, B1bn, B2bn, M0bn, M1bn, M2bn,
                W1bn, W2bn, g4b_n, awb_n)

    lax.fori_loop(0, S, body, init)


def _tc_lattice(emb, gefw, gebw, valfw, valbw,
                fwWh, fwWwx, fwWwh, fwbwb, fwWlc,
                bwWh, bwWwx, bwWwh, bwbwb, bwWlc,
                fwWx, fwb, fwWlx, fwbl, bwWx, bwb, bwWlx, bwbl):
    return pl.pallas_call(
        _tc_body,
        out_shape=jax.ShapeDtypeStruct((S, 2 * H), _F32),
        scratch_shapes=[
            pltpu.VMEM((S, 4 * H), _F32),
            pltpu.VMEM((S, H), _F32),
            pltpu.VMEM((S, 4 * H), _F32),
            pltpu.VMEM((S, H), _F32),
            pltpu.VMEM((S * 8, 3 * H), _F32),
            pltpu.VMEM((S * 8, 3 * H), _F32),
        ],
    )(emb, gefw, gebw, valfw, valbw,
      fwWh, fwWwx, fwWwh, fwbwb, fwWlc,
      bwWh, bwWwx, bwWwh, bwbwb, bwWlc,
      fwWx, fwb, fwWlx, fwbl, bwWx, bwb, bwWlx, bwbl)


def kernel(tokens, gaz_ids, gaz_lengths, token_table, gaz_table,
           fw_Wx, fw_Wh, fw_b, fw_Wwx, fw_Wwh, fw_bw, fw_Wlx, fw_Wlc, fw_bl,
           bw_Wx, bw_Wh, bw_b, bw_Wwx, bw_Wwh, bw_bw, bw_Wlx, bw_Wlc, bw_bl):
    tok_idx = tokens.reshape(S).astype(jnp.int32)
    gi = gaz_ids.astype(jnp.int32)
    gl = gaz_lengths.astype(jnp.int32)
    pos = jnp.arange(S, dtype=jnp.int32)[:, None]


    bw_cols, vf_cols, vb_cols = [], [], []
    for dd in (1, 2, 3):
        gi_s = jnp.concatenate([jnp.zeros((dd, MAXG), jnp.int32), gi[:S - dd]], axis=0)
        gl_s = jnp.concatenate([jnp.zeros((dd, MAXG), jnp.int32), gl[:S - dd]], axis=0)
        bw_cols.append(gi_s)
        vf_cols.append((gl == dd) & (pos + dd < S))
        vb_cols.append((pos >= dd) & (gl_s == dd))
    zpad2 = jnp.zeros((S, 2), jnp.int32)
    fw_idx = jnp.concatenate([gi, jnp.zeros((S, 6), jnp.int32)], axis=1).reshape(S * 8)
    bw_idx = jnp.concatenate(bw_cols + [zpad2], axis=1).reshape(S * 8)
    zpadb = jnp.zeros((S, 2), bool)
    valfw = jnp.broadcast_to(
        jnp.concatenate(vf_cols + [zpadb], axis=1).astype(_F32).reshape(S * 8, 1),
        (S * 8, H))
    valbw = jnp.broadcast_to(
        jnp.concatenate(vb_cols + [zpadb], axis=1).astype(_F32).reshape(S * 8, 1),
        (S * 8, H))

    emb, gefw, gebw = _sc_gather(token_table, gaz_table, tok_idx, fw_idx, bw_idx)

    bf = jnp.bfloat16
    hs = _tc_lattice(
        emb, gefw, gebw, valfw, valbw,
        fw_Wh.astype(bf), fw_Wwx.astype(bf), fw_Wwh.astype(bf),
        fw_bw.reshape(1, 3 * H), fw_Wlc.astype(bf),
        bw_Wh.astype(bf), bw_Wwx.astype(bf), bw_Wwh.astype(bf),
        bw_bw.reshape(1, 3 * H), bw_Wlc.astype(bf),
        fw_Wx, fw_b.reshape(1, 4 * H), fw_Wlx, fw_bl.reshape(1, H),
        bw_Wx, bw_b.reshape(1, 4 * H), bw_Wlx, bw_bl.reshape(1, H))
    return hs[None, :, :]


# speedup vs baseline: 1.4391x; 1.4391x over previous
import functools

import jax
import jax.numpy as jnp
from jax import lax
from jax.experimental import pallas as pl
from jax.experimental.pallas import tpu as pltpu
from jax.experimental.pallas import tpu_sc as plsc

S = 512
H = 256
D = 128
MAXG = 2

_F32 = jnp.float32


def _sc_gather(token_table, gaz_table, tok_idx, gaz_idx):
    mesh = plsc.VectorSubcoreMesh(core_axis_name="c", subcore_axis_name="s")

    @functools.partial(
        pl.kernel,
        mesh=mesh,
        out_type=[
            jax.ShapeDtypeStruct((S, D), _F32),
            jax.ShapeDtypeStruct((S * MAXG, D), _F32),
        ],
        scratch_types=[
            pltpu.VMEM((16,), jnp.int32),
            pltpu.VMEM((16, D), _F32),
            pltpu.VMEM((32,), jnp.int32),
            pltpu.VMEM((32, D), _F32),
            pltpu.SemaphoreType.DMA,
            pltpu.SemaphoreType.DMA,
        ],
    )
    def gk(tok_tab, gaz_tab, t_idx, g_idx, emb_o, gw_o,
           ti_v, tr_v, gi_v, gr_v, sem1, sem2):
        wid = lax.axis_index("s") * 2 + lax.axis_index("c")
        pltpu.sync_copy(t_idx.at[pl.ds(wid * 16, 16)], ti_v)
        pltpu.sync_copy(g_idx.at[pl.ds(wid * 32, 32)], gi_v)
        cp1 = pltpu.async_copy(tok_tab.at[ti_v], tr_v, sem1)
        cp2 = pltpu.async_copy(gaz_tab.at[gi_v], gr_v, sem2)
        cp1.wait()
        pltpu.sync_copy(tr_v, emb_o.at[pl.ds(wid * 16, 16)])
        cp2.wait()
        pltpu.sync_copy(gr_v, gw_o.at[pl.ds(wid * 32, 32)])

    return gk(token_table, gaz_table, tok_idx, gaz_idx)


def _dot(a, b):
    return jnp.dot(a, b, preferred_element_type=_F32)


def _dotb(a, b):
    return jnp.dot(a.astype(jnp.bfloat16), b, preferred_element_type=_F32)


def _shift3(A1, A2, new6):
    z4 = jnp.zeros((4, H), _F32)
    A0n = jnp.concatenate([A1[0:4], new6[0:2]], axis=0)
    A1n = jnp.concatenate([A2[0:2], new6[2:4], A2[4:6]], axis=0)
    A2n = jnp.concatenate([new6[4:6], z4], axis=0)
    return A0n, A1n, A2n


def _shift3r(A1, A2, new6r):
    z4 = jnp.zeros((4, H), _F32)
    A0n = jnp.concatenate([A1[0:4], new6r[4:6]], axis=0)
    A1n = jnp.concatenate([A2[0:2], new6r[2:4], A2[4:6]], axis=0)
    A2n = jnp.concatenate([new6r[0:2], z4], axis=0)
    return A0n, A1n, A2n


def _gates(g4):
    sg = jax.nn.sigmoid(g4[:, :3 * H])
    return sg[:, :H], sg[:, H:2 * H], sg[:, 2 * H:], jnp.tanh(g4[:, 3 * H:])


def _cnew(c, B0, M0, aw, i_g, f_g, g_g):
    ew = M0 * jnp.exp(jax.nn.sigmoid(aw))
    e0 = jnp.exp(i_g)
    s_e = jnp.sum(ew, axis=0, keepdims=True)
    s_ec = jnp.sum(ew * B0, axis=0, keepdims=True)
    anym = jnp.max(M0, axis=0, keepdims=True)
    c_multi = (e0 * g_g + s_ec) / (e0 + s_e)
    c_plain = f_g * c + i_g * g_g
    return jnp.where(anym > 0.5, c_multi, c_plain)


def _wordcells(wg, c_new):
    sg = jax.nn.sigmoid(wg[:, :2 * H])
    iw, fw_, gw = sg[:, :H], sg[:, H:], jnp.tanh(wg[:, 2 * H:])
    return fw_ * c_new + iw * gw


def _tc_body(emb, gw_pad, vbf_ref, vbb_ref,
             fwWh, fwWwx, fwWwh, fwbwb, fwWlc,
             bwWh, bwWwx, bwWwh, bwbwb, bwWlc,
             fwWx, fwb, fwWlx, fwbl, bwWx, bwb, bwWlx, bwbl,
             hs_ref, xwf, xlf, xwb, xlb):
    for ci in range(8):
        r0 = ci * 64
        e = emb[r0:r0 + 64, :]
        xwf[r0:r0 + 64, :] = _dot(e, fwWx[:, :]) + fwb[:, :]
        xlf[r0:r0 + 64, :] = _dot(e, fwWlx[:, :]) + fwbl[:, :]
        xwb[r0:r0 + 64, :] = _dot(e, bwWx[:, :]) + bwb[:, :]
        xlb[r0:r0 + 64, :] = _dot(e, bwWlx[:, :]) + bwbl[:, :]

    z1 = jnp.zeros((1, H), _F32)
    z6 = jnp.zeros((6, H), _F32)
    init = (z1, z1, z6, z6, z6, z6, z6, z6, z6, z6,
            xwf[0:1, :], jnp.broadcast_to(xlf[0:1, :], (6, H)),
            z1, z1, z6, z6, z6, z6, z6, z6, z6, z6,
            xwb[S - 1:S, :], jnp.broadcast_to(xlb[S - 1:S, :], (6, H)))

    def body(t, carry):
        (hf, cf, B0f, B1f, B2f, M0f, M1f, M2f, W1f, W2f, g4f, awf,
         hb, cb, B0b, B1b, B2b, M0b, M1b, M2b, W1b, W2b, g4b, awb) = carry
        p = S - 1 - t
        tn = jnp.minimum(t + 1, S - 1)
        pn = jnp.maximum(p - 1, 0)
        if_, ff, of, gf = _gates(g4f)
        ib_, fb, ob, gb = _gates(g4b)
        cf_n = _cnew(cf, B0f, M0f, awf, if_, ff, gf)
        cb_n = _cnew(cb, B0b, M0b, awb, ib_, fb, gb)
        hf_n = of * jnp.tanh(cf_n)
        hb_n = ob * jnp.tanh(cb_n)
        hs_ref[pl.ds(t, 1), 0:H] = hf_n
        hs_ref[pl.ds(p, 1), H:2 * H] = hb_n
        ge_f = gw_pad[pl.ds(2 * t + 6, 2), :]
        ge_b = gw_pad[pl.ds(2 * p, 6), :]
        vb_f = vbf_ref[pl.ds(8 * t, 8), :][0:6]
        vb_b = vbb_ref[pl.ds(8 * p, 8), :][0:6]
        wgf = _dotb(ge_f, fwWwx[:, :]) + fwbwb[:, :] + _dotb(hf_n, fwWwh[:, :])
        wgb = _dotb(ge_b, bwWwx[:, :]) + bwbwb[:, :] + _dotb(hb_n, bwWwh[:, :])
        cwf = _wordcells(wgf, cf_n)
        cwb = _wordcells(wgb, cb_n)
        cwWf = _dotb(cwf, fwWlc[:, :])
        cwWb = _dotb(cwb, bwWlc[:, :])
        cw6f = jnp.concatenate([cwf, cwf, cwf], axis=0)
        cwW6f = jnp.concatenate([cwWf, cwWf, cwWf], axis=0)
        B0fn, B1fn, B2fn = _shift3(B1f, B2f, cw6f)
        B0bn, B1bn, B2bn = _shift3r(B1b, B2b, cwb)
        M0fn, M1fn, M2fn = _shift3(M1f, M2f, vb_f)
        M0bn, M1bn, M2bn = _shift3r(M1b, M2b, vb_b)
        W0fn, W1fn, W2fn = _shift3(W1f, W2f, cwW6f)
        W0bn, W1bn, W2bn = _shift3r(W1b, W2b, cwWb)
        g4f_n = xwf[pl.ds(tn, 1), :] + _dotb(hf_n, fwWh[:, :])
        g4b_n = xwb[pl.ds(pn, 1), :] + _dotb(hb_n, bwWh[:, :])
        awf_n = xlf[pl.ds(tn, 1), :] + W0fn
        awb_n = xlb[pl.ds(pn, 1), :] + W0bn
        return (hf_n, cf_n, B0fn, B1fn, B2fn, M0fn, M1fn, M2fn,
                W1fn, W2fn, g4f_n, awf_n,
                hb_n, cb_n, B0bn, B1bn, B2bn, M0bn, M1bn, M2bn,
                W1bn, W2bn, g4b_n, awb_n)

    lax.fori_loop(0, S, body, init)


def _tc_lattice(emb, gw_pad, valfw, valbw,
                fwWh, fwWwx, fwWwh, fwbwb, fwWlc,
                bwWh, bwWwx, bwWwh, bwbwb, bwWlc,
                fwWx, fwb, fwWlx, fwbl, bwWx, bwb, bwWlx, bwbl):
    return pl.pallas_call(
        _tc_body,
        out_shape=jax.ShapeDtypeStruct((S, 2 * H), _F32),
        scratch_shapes=[
            pltpu.VMEM((S, 4 * H), _F32),
            pltpu.VMEM((S, H), _F32),
            pltpu.VMEM((S, 4 * H), _F32),
            pltpu.VMEM((S, H), _F32),
        ],
    )(emb, gw_pad, valfw, valbw,
      fwWh, fwWwx, fwWwh, fwbwb, fwWlc,
      bwWh, bwWwx, bwWwh, bwbwb, bwWlc,
      fwWx, fwb, fwWlx, fwbl, bwWx, bwb, bwWlx, bwbl)


def kernel(tokens, gaz_ids, gaz_lengths, token_table, gaz_table,
           fw_Wx, fw_Wh, fw_b, fw_Wwx, fw_Wwh, fw_bw, fw_Wlx, fw_Wlc, fw_bl,
           bw_Wx, bw_Wh, bw_b, bw_Wwx, bw_Wwh, bw_bw, bw_Wlx, bw_Wlc, bw_bl):
    tok_idx = tokens.reshape(S).astype(jnp.int32)
    gi = gaz_ids.astype(jnp.int32)
    gl = gaz_lengths.astype(jnp.int32)
    pos = jnp.arange(S, dtype=jnp.int32)[:, None]


    gaz_idx = gi.reshape(S * MAXG)

    vf_cols, vb_cols = [], []
    for dd in (1, 2, 3):
        gl_s = jnp.concatenate([jnp.zeros((dd, MAXG), jnp.int32), gl[:S - dd]], axis=0)
        vf_cols.append((gl == dd) & (pos + dd < S))
        vb_cols.insert(0, (pos >= dd) & (gl_s == dd))
    zpadb = jnp.zeros((S, 2), bool)
    valfw = jnp.broadcast_to(
        jnp.concatenate(vf_cols + [zpadb], axis=1).astype(_F32).reshape(S * 8, 1),
        (S * 8, H))
    valbw = jnp.broadcast_to(
        jnp.concatenate(vb_cols + [zpadb], axis=1).astype(_F32).reshape(S * 8, 1),
        (S * 8, H))

    emb, gw = _sc_gather(token_table, gaz_table, tok_idx, gaz_idx)
    gw_pad = jnp.concatenate(
        [jnp.zeros((6, D), _F32), gw, jnp.zeros((2, D), _F32)], axis=0)

    bf = jnp.bfloat16
    hs = _tc_lattice(
        emb, gw_pad, valfw, valbw,
        fw_Wh.astype(bf), fw_Wwx.astype(bf), fw_Wwh.astype(bf),
        fw_bw.reshape(1, 3 * H), fw_Wlc.astype(bf),
        bw_Wh.astype(bf), bw_Wwx.astype(bf), bw_Wwh.astype(bf),
        bw_bw.reshape(1, 3 * H), bw_Wlc.astype(bf),
        fw_Wx, fw_b.reshape(1, 4 * H), fw_Wlx, fw_bl.reshape(1, H),
        bw_Wx, bw_b.reshape(1, 4 * H), bw_Wlx, bw_bl.reshape(1, H))
    return hs[None, :, :]


# speedup vs baseline: 1.5219x; 1.0575x over previous
import functools

import jax
import jax.numpy as jnp
from jax import lax
from jax.experimental import pallas as pl
from jax.experimental.pallas import tpu as pltpu
from jax.experimental.pallas import tpu_sc as plsc

S = 512
H = 256
D = 128
MAXG = 2

_F32 = jnp.float32


def _sc_gather(token_table, gaz_table, tok_idx, gaz_idx):
    mesh = plsc.VectorSubcoreMesh(core_axis_name="c", subcore_axis_name="s")

    @functools.partial(
        pl.kernel,
        mesh=mesh,
        out_type=[
            jax.ShapeDtypeStruct((S, D), _F32),
            jax.ShapeDtypeStruct((S * MAXG, D), _F32),
        ],
        scratch_types=[
            pltpu.VMEM((16,), jnp.int32),
            pltpu.VMEM((16, D), _F32),
            pltpu.VMEM((32,), jnp.int32),
            pltpu.VMEM((32, D), _F32),
            pltpu.SemaphoreType.DMA,
            pltpu.SemaphoreType.DMA,
        ],
    )
    def gk(tok_tab, gaz_tab, t_idx, g_idx, emb_o, gw_o,
           ti_v, tr_v, gi_v, gr_v, sem1, sem2):
        wid = lax.axis_index("s") * 2 + lax.axis_index("c")
        pltpu.sync_copy(t_idx.at[pl.ds(wid * 16, 16)], ti_v)
        pltpu.sync_copy(g_idx.at[pl.ds(wid * 32, 32)], gi_v)
        cp1 = pltpu.async_copy(tok_tab.at[ti_v], tr_v, sem1)
        cp2 = pltpu.async_copy(gaz_tab.at[gi_v], gr_v, sem2)
        cp1.wait()
        pltpu.sync_copy(tr_v, emb_o.at[pl.ds(wid * 16, 16)])
        cp2.wait()
        pltpu.sync_copy(gr_v, gw_o.at[pl.ds(wid * 32, 32)])

    return gk(token_table, gaz_table, tok_idx, gaz_idx)


def _dot(a, b):
    return jnp.dot(a, b, preferred_element_type=_F32)


def _dotb(a, b):
    return jnp.dot(a.astype(jnp.bfloat16), b, preferred_element_type=_F32)


def _shift3(A1, A2, new6):
    z4 = jnp.zeros((4, H), _F32)
    A0n = jnp.concatenate([A1[0:4], new6[0:2]], axis=0)
    A1n = jnp.concatenate([A2[0:2], new6[2:4], A2[4:6]], axis=0)
    A2n = jnp.concatenate([new6[4:6], z4], axis=0)
    return A0n, A1n, A2n


def _shift3r(A1, A2, new6r):
    z4 = jnp.zeros((4, H), _F32)
    A0n = jnp.concatenate([A1[0:4], new6r[4:6]], axis=0)
    A1n = jnp.concatenate([A2[0:2], new6r[2:4], A2[4:6]], axis=0)
    A2n = jnp.concatenate([new6r[0:2], z4], axis=0)
    return A0n, A1n, A2n


def _gates(g4):
    sg = jax.nn.sigmoid(g4[:, :3 * H])
    return sg[:, :H], sg[:, H:2 * H], sg[:, 2 * H:], jnp.tanh(g4[:, 3 * H:])


def _cnew(c, B0, M0, aw, i_g, f_g, g_g):
    ew = M0 * jnp.exp(jax.nn.sigmoid(aw))
    e0 = jnp.exp(i_g)
    s_e = jnp.sum(ew, axis=0, keepdims=True)
    s_ec = jnp.sum(ew * B0, axis=0, keepdims=True)
    anym = jnp.max(M0, axis=0, keepdims=True)
    c_multi = (e0 * g_g + s_ec) / (e0 + s_e)
    c_plain = f_g * c + i_g * g_g
    return jnp.where(anym > 0.5, c_multi, c_plain)


def _wordcells(wg, c_new):
    sg = jax.nn.sigmoid(wg[:, :2 * H])
    iw, fw_, gw = sg[:, :H], sg[:, H:], jnp.tanh(wg[:, 2 * H:])
    return fw_ * c_new + iw * gw


def _dir_step(tn_idx, st, gx, vb, xw_ref, xl_ref, Wh, Wwh, Wlc, rev):
    (h, c, B0, B1, B2, M0, M1, M2, W1, W2, g4, aw) = st
    i_g, f_g, o_g, g_g = _gates(g4)
    c_n = _cnew(c, B0, M0, aw, i_g, f_g, g_g)
    h_n = o_g * jnp.tanh(c_n)
    wg = gx + _dotb(h_n, Wwh[:, :])
    cw = _wordcells(wg, c_n)
    cwW = _dotb(cw, Wlc[:, :])
    if rev:
        B0n, B1n, B2n = _shift3r(B1, B2, cw)
        M0n, M1n, M2n = _shift3r(M1, M2, vb)
        W0n, W1n, W2n = _shift3r(W1, W2, cwW)
    else:
        cw6 = jnp.concatenate([cw, cw, cw], axis=0)
        cwW6 = jnp.concatenate([cwW, cwW, cwW], axis=0)
        B0n, B1n, B2n = _shift3(B1, B2, cw6)
        M0n, M1n, M2n = _shift3(M1, M2, vb)
        W0n, W1n, W2n = _shift3(W1, W2, cwW6)
    g4n = xw_ref[pl.ds(tn_idx, 1), :] + _dotb(h_n, Wh[:, :])
    awn = xl_ref[pl.ds(tn_idx, 1), :] + W0n
    return (h_n, c_n, B0n, B1n, B2n, M0n, M1n, M2n, W1n, W2n, g4n, awn)


def _tc_body(emb, gw_pad, vbf_ref, vbb_ref,
             fwWh, fwWwx, fwWwh, fwbwb, fwWlc,
             bwWh, bwWwx, bwWwh, bwbwb, bwWlc,
             fwWx, fwb, fwWlx, fwbl, bwWx, bwb, bwWlx, bwbl,
             hs_ref, xwf, xlf, xwb, xlb, gxf, gxb):
    for ci in range(8):
        r0 = ci * 64
        e = emb[r0:r0 + 64, :]
        xwf[r0:r0 + 64, :] = _dot(e, fwWx[:, :]) + fwb[:, :]
        xlf[r0:r0 + 64, :] = _dot(e, fwWlx[:, :]) + fwbl[:, :]
        xwb[r0:r0 + 64, :] = _dot(e, bwWx[:, :]) + bwb[:, :]
        xlb[r0:r0 + 64, :] = _dot(e, bwWlx[:, :]) + bwbl[:, :]
    for ci in range(5):
        r0 = ci * 256
        g = gw_pad[r0:r0 + 256, :]
        gxf[r0:r0 + 256, :] = _dotb(g, fwWwx[:, :]) + fwbwb[:, :]
        gxb[r0:r0 + 256, :] = _dotb(g, bwWwx[:, :]) + bwbwb[:, :]

    z1 = jnp.zeros((1, H), _F32)
    z6 = jnp.zeros((6, H), _F32)
    init = (z1, z1, z6, z6, z6, z6, z6, z6, z6, z6,
            xwf[0:1, :], jnp.broadcast_to(xlf[0:1, :], (6, H)),
            z1, z1, z6, z6, z6, z6, z6, z6, z6, z6,
            xwb[S - 1:S, :], jnp.broadcast_to(xlb[S - 1:S, :], (6, H)))

    def body(u, carry):
        stf = carry[:12]
        stb = carry[12:]
        blkGF = gxf[pl.ds(8 * u, 16), :]
        blkGB = gxb[pl.ds(1016 - 8 * u, 16), :]
        blkVF = vbf_ref[pl.ds(32 * u, 32), :]
        blkVB = vbb_ref[pl.ds(4064 - 32 * u, 32), :]
        for i in range(4):
            t = 4 * u + i
            p = S - 1 - t
            tn = jnp.minimum(t + 1, S - 1)
            pn = jnp.maximum(p - 1, 0)
            stf = _dir_step(tn, stf, blkGF[6 + 2 * i:8 + 2 * i],
                            blkVF[8 * i:8 * i + 6], xwf, xlf,
                            fwWh, fwWwh, fwWlc, False)
            stb = _dir_step(pn, stb, blkGB[6 - 2 * i:12 - 2 * i],
                            blkVB[24 - 8 * i:30 - 8 * i], xwb, xlb,
                            bwWh, bwWwh, bwWlc, True)
            hs_ref[pl.ds(t, 1), 0:H] = stf[0]
            hs_ref[pl.ds(p, 1), H:2 * H] = stb[0]
        return (*stf, *stb)

    lax.fori_loop(0, S // 4, body, init)


def _tc_lattice(emb, gw_pad, valfw, valbw,
                fwWh, fwWwx, fwWwh, fwbwb, fwWlc,
                bwWh, bwWwx, bwWwh, bwbwb, bwWlc,
                fwWx, fwb, fwWlx, fwbl, bwWx, bwb, bwWlx, bwbl):
    return pl.pallas_call(
        _tc_body,
        out_shape=jax.ShapeDtypeStruct((S, 2 * H), _F32),
        scratch_shapes=[
            pltpu.VMEM((S, 4 * H), _F32),
            pltpu.VMEM((S, H), _F32),
            pltpu.VMEM((S, 4 * H), _F32),
            pltpu.VMEM((S, H), _F32),
            pltpu.VMEM((1280, 3 * H), _F32),
            pltpu.VMEM((1280, 3 * H), _F32),
        ],
    )(emb, gw_pad, valfw, valbw,
      fwWh, fwWwx, fwWwh, fwbwb, fwWlc,
      bwWh, bwWwx, bwWwh, bwbwb, bwWlc,
      fwWx, fwb, fwWlx, fwbl, bwWx, bwb, bwWlx, bwbl)


def kernel(tokens, gaz_ids, gaz_lengths, token_table, gaz_table,
           fw_Wx, fw_Wh, fw_b, fw_Wwx, fw_Wwh, fw_bw, fw_Wlx, fw_Wlc, fw_bl,
           bw_Wx, bw_Wh, bw_b, bw_Wwx, bw_Wwh, bw_bw, bw_Wlx, bw_Wlc, bw_bl):
    tok_idx = tokens.reshape(S).astype(jnp.int32)
    gi = gaz_ids.astype(jnp.int32)
    gl = gaz_lengths.astype(jnp.int32)
    pos = jnp.arange(S, dtype=jnp.int32)[:, None]


    gaz_idx = gi.reshape(S * MAXG)

    vf_cols, vb_cols = [], []
    for dd in (1, 2, 3):
        gl_s = jnp.concatenate([jnp.zeros((dd, MAXG), jnp.int32), gl[:S - dd]], axis=0)
        vf_cols.append((gl == dd) & (pos + dd < S))
        vb_cols.insert(0, (pos >= dd) & (gl_s == dd))
    zpadb = jnp.zeros((S, 2), bool)
    valfw = jnp.broadcast_to(
        jnp.concatenate(vf_cols + [zpadb], axis=1).astype(_F32).reshape(S * 8, 1),
        (S * 8, H))
    valbw = jnp.broadcast_to(
        jnp.concatenate(vb_cols + [zpadb], axis=1).astype(_F32).reshape(S * 8, 1),
        (S * 8, H))

    emb, gw = _sc_gather(token_table, gaz_table, tok_idx, gaz_idx)
    gw_pad = jnp.concatenate(
        [jnp.zeros((6, D), _F32), gw, jnp.zeros((250, D), _F32)], axis=0)

    bf = jnp.bfloat16
    hs = _tc_lattice(
        emb, gw_pad, valfw, valbw,
        fw_Wh.astype(bf), fw_Wwx.astype(bf), fw_Wwh.astype(bf),
        fw_bw.reshape(1, 3 * H), fw_Wlc.astype(bf),
        bw_Wh.astype(bf), bw_Wwx.astype(bf), bw_Wwh.astype(bf),
        bw_bw.reshape(1, 3 * H), bw_Wlc.astype(bf),
        fw_Wx, fw_b.reshape(1, 4 * H), fw_Wlx, fw_bl.reshape(1, H),
        bw_Wx, bw_b.reshape(1, 4 * H), bw_Wlx, bw_bl.reshape(1, H))
    return hs[None, :, :]


# speedup vs baseline: 1.9780x; 1.2997x over previous
import functools

import jax
import jax.numpy as jnp
from jax import lax
from jax.experimental import pallas as pl
from jax.experimental.pallas import tpu as pltpu
from jax.experimental.pallas import tpu_sc as plsc

S = 512
H = 256
D = 128
MAXG = 2

_F32 = jnp.float32


def _sc_gather(token_table, gaz_table, tok_idx, gaz_idx):
    mesh = plsc.VectorSubcoreMesh(core_axis_name="c", subcore_axis_name="s")

    @functools.partial(
        pl.kernel,
        mesh=mesh,
        out_type=[
            jax.ShapeDtypeStruct((S, D), _F32),
            jax.ShapeDtypeStruct((S * MAXG, D), _F32),
        ],
        scratch_types=[
            pltpu.VMEM((16,), jnp.int32),
            pltpu.VMEM((16, D), _F32),
            pltpu.VMEM((32,), jnp.int32),
            pltpu.VMEM((32, D), _F32),
            pltpu.SemaphoreType.DMA,
            pltpu.SemaphoreType.DMA,
        ],
    )
    def gk(tok_tab, gaz_tab, t_idx, g_idx, emb_o, gw_o,
           ti_v, tr_v, gi_v, gr_v, sem1, sem2):
        wid = lax.axis_index("s") * 2 + lax.axis_index("c")
        pltpu.sync_copy(t_idx.at[pl.ds(wid * 16, 16)], ti_v)
        pltpu.sync_copy(g_idx.at[pl.ds(wid * 32, 32)], gi_v)
        cp1 = pltpu.async_copy(tok_tab.at[ti_v], tr_v, sem1)
        cp2 = pltpu.async_copy(gaz_tab.at[gi_v], gr_v, sem2)
        cp1.wait()
        pltpu.sync_copy(tr_v, emb_o.at[pl.ds(wid * 16, 16)])
        cp2.wait()
        pltpu.sync_copy(gr_v, gw_o.at[pl.ds(wid * 32, 32)])

    return gk(token_table, gaz_table, tok_idx, gaz_idx)


def _dot(a, b):
    return jnp.dot(a, b, preferred_element_type=_F32)


def _dotb(a, b):
    return jnp.dot(a.astype(jnp.bfloat16), b, preferred_element_type=_F32)


def _shift3(A1, A2, new6):
    z4 = jnp.zeros((4, H), _F32)
    A0n = jnp.concatenate([A1[0:4], new6[0:2]], axis=0)
    A1n = jnp.concatenate([A2[0:2], new6[2:4], A2[4:6]], axis=0)
    A2n = jnp.concatenate([new6[4:6], z4], axis=0)
    return A0n, A1n, A2n


def _shift3r(A1, A2, new6r):
    z4 = jnp.zeros((4, H), _F32)
    A0n = jnp.concatenate([A1[0:4], new6r[4:6]], axis=0)
    A1n = jnp.concatenate([A2[0:2], new6r[2:4], A2[4:6]], axis=0)
    A2n = jnp.concatenate([new6r[0:2], z4], axis=0)
    return A0n, A1n, A2n


def _gates(g4):
    sg = jax.nn.sigmoid(g4[:, :3 * H])
    return sg[:, :H], sg[:, H:2 * H], sg[:, 2 * H:], jnp.tanh(g4[:, 3 * H:])


def _cnew(c, B0, M0, aw, i_g, f_g, g_g):
    ew = M0 * jnp.exp(jax.nn.sigmoid(aw))
    e0 = jnp.exp(i_g)
    s_e = jnp.sum(ew, axis=0, keepdims=True)
    s_ec = jnp.sum(ew * B0, axis=0, keepdims=True)
    anym = jnp.max(M0, axis=0, keepdims=True)
    c_multi = (e0 * g_g + s_ec) / (e0 + s_e)
    c_plain = f_g * c + i_g * g_g
    return jnp.where(anym > 0.5, c_multi, c_plain)


def _wordcells(wg, c_new):
    sg = jax.nn.sigmoid(wg[:, :2 * H])
    iw, fw_, gw = sg[:, :H], sg[:, H:], jnp.tanh(wg[:, 2 * H:])
    return fw_ * c_new + iw * gw


def _dir_step(tn_idx, st, gx, vb, xw_ref, xl_ref, Wh, Wwh, Wlc, rev):
    (h, c, B0, B1, B2, M0, M1, M2, g4, aw) = st
    i_g, f_g, o_g, g_g = _gates(g4)
    c_n = _cnew(c, B0, M0, aw, i_g, f_g, g_g)
    h_n = o_g * jnp.tanh(c_n)
    wg = gx + _dotb(h_n, Wwh[:, :])
    cw = _wordcells(wg, c_n)
    if rev:
        B0n, B1n, B2n = _shift3r(B1, B2, cw)
        M0n, M1n, M2n = _shift3r(M1, M2, vb)
        cw_dd1 = cw[4:6]
    else:
        cw6 = jnp.concatenate([cw, cw, cw], axis=0)
        B0n, B1n, B2n = _shift3(B1, B2, cw6)
        M0n, M1n, M2n = _shift3(M1, M2, vb)
        cw_dd1 = cw[0:2]
    g4n = xw_ref[pl.ds(tn_idx, 1), :] + _dotb(h_n, Wh[:, :])
    awn = xl_ref[pl.ds(tn_idx, 1), :] + jnp.concatenate(
        [_dotb(B1[0:4], Wlc[:, :]), _dotb(cw_dd1, Wlc[:, :])], axis=0)
    return (h_n, c_n, B0n, B1n, B2n, M0n, M1n, M2n, g4n, awn)


def _tc_body(emb, gw_pad, vbf_ref, vbb_ref,
             fwWh, fwWwx, fwWwh, fwbwb, fwWlc,
             bwWh, bwWwx, bwWwh, bwbwb, bwWlc,
             fwWx, fwb, fwWlx, fwbl, bwWx, bwb, bwWlx, bwbl,
             hs_ref, xwf, xlf, xwb, xlb, gxf, gxb):
    for ci in range(8):
        r0 = ci * 64
        e = emb[r0:r0 + 64, :]
        xwf[r0:r0 + 64, :] = _dot(e, fwWx[:, :]) + fwb[:, :]
        xlf[r0:r0 + 64, :] = _dot(e, fwWlx[:, :]) + fwbl[:, :]
        xwb[r0:r0 + 64, :] = _dot(e, bwWx[:, :]) + bwb[:, :]
        xlb[r0:r0 + 64, :] = _dot(e, bwWlx[:, :]) + bwbl[:, :]
    for ci in range(5):
        r0 = ci * 256
        g = gw_pad[r0:r0 + 256, :]
        gxf[r0:r0 + 256, :] = _dotb(g, fwWwx[:, :]) + fwbwb[:, :]
        gxb[r0:r0 + 256, :] = _dotb(g, bwWwx[:, :]) + bwbwb[:, :]

    z1 = jnp.zeros((1, H), _F32)
    z6 = jnp.zeros((6, H), _F32)
    init = (z1, z1, z6, z6, z6, z6, z6, z6,
            xwf[0:1, :], jnp.broadcast_to(xlf[0:1, :], (6, H)),
            z1, z1, z6, z6, z6, z6, z6, z6,
            xwb[S - 1:S, :], jnp.broadcast_to(xlb[S - 1:S, :], (6, H)))

    def body(u, carry):
        stf = carry[:10]
        stb = carry[10:]
        blkGF = gxf[pl.ds(8 * u, 16), :]
        blkGB = gxb[pl.ds(1016 - 8 * u, 16), :]
        blkVF = vbf_ref[pl.ds(32 * u, 32), :]
        blkVB = vbb_ref[pl.ds(4064 - 32 * u, 32), :]
        for i in range(4):
            t = 4 * u + i
            p = S - 1 - t
            tn = jnp.minimum(t + 1, S - 1)
            pn = jnp.maximum(p - 1, 0)
            stf = _dir_step(tn, stf, blkGF[6 + 2 * i:8 + 2 * i],
                            blkVF[8 * i:8 * i + 6], xwf, xlf,
                            fwWh, fwWwh, fwWlc, False)
            stb = _dir_step(pn, stb, blkGB[6 - 2 * i:12 - 2 * i],
                            blkVB[24 - 8 * i:30 - 8 * i], xwb, xlb,
                            bwWh, bwWwh, bwWlc, True)
            hs_ref[pl.ds(t, 1), 0:H] = stf[0]
            hs_ref[pl.ds(p, 1), H:2 * H] = stb[0]
        return (*stf, *stb)

    lax.fori_loop(0, S // 4, body, init)


def _tc_lattice(emb, gw_pad, valfw, valbw,
                fwWh, fwWwx, fwWwh, fwbwb, fwWlc,
                bwWh, bwWwx, bwWwh, bwbwb, bwWlc,
                fwWx, fwb, fwWlx, fwbl, bwWx, bwb, bwWlx, bwbl):
    return pl.pallas_call(
        _tc_body,
        out_shape=jax.ShapeDtypeStruct((S, 2 * H), _F32),
        scratch_shapes=[
            pltpu.VMEM((S, 4 * H), _F32),
            pltpu.VMEM((S, H), _F32),
            pltpu.VMEM((S, 4 * H), _F32),
            pltpu.VMEM((S, H), _F32),
            pltpu.VMEM((1280, 3 * H), _F32),
            pltpu.VMEM((1280, 3 * H), _F32),
        ],
    )(emb, gw_pad, valfw, valbw,
      fwWh, fwWwx, fwWwh, fwbwb, fwWlc,
      bwWh, bwWwx, bwWwh, bwbwb, bwWlc,
      fwWx, fwb, fwWlx, fwbl, bwWx, bwb, bwWlx, bwbl)


def kernel(tokens, gaz_ids, gaz_lengths, token_table, gaz_table,
           fw_Wx, fw_Wh, fw_b, fw_Wwx, fw_Wwh, fw_bw, fw_Wlx, fw_Wlc, fw_bl,
           bw_Wx, bw_Wh, bw_b, bw_Wwx, bw_Wwh, bw_bw, bw_Wlx, bw_Wlc, bw_bl):
    tok_idx = tokens.reshape(S).astype(jnp.int32)
    gi = gaz_ids.astype(jnp.int32)
    gl = gaz_lengths.astype(jnp.int32)
    pos = jnp.arange(S, dtype=jnp.int32)[:, None]


    gaz_idx = gi.reshape(S * MAXG)

    vf_cols, vb_cols = [], []
    for dd in (1, 2, 3):
        gl_s = jnp.concatenate([jnp.zeros((dd, MAXG), jnp.int32), gl[:S - dd]], axis=0)
        vf_cols.append((gl == dd) & (pos + dd < S))
        vb_cols.insert(0, (pos >= dd) & (gl_s == dd))
    zpadb = jnp.zeros((S, 2), bool)
    valfw = jnp.broadcast_to(
        jnp.concatenate(vf_cols + [zpadb], axis=1).astype(_F32).reshape(S * 8, 1),
        (S * 8, H))
    valbw = jnp.broadcast_to(
        jnp.concatenate(vb_cols + [zpadb], axis=1).astype(_F32).reshape(S * 8, 1),
        (S * 8, H))

    emb, gw = _sc_gather(token_table, gaz_table, tok_idx, gaz_idx)
    gw_pad = jnp.concatenate(
        [jnp.zeros((6, D), _F32), gw, jnp.zeros((250, D), _F32)], axis=0)

    bf = jnp.bfloat16
    hs = _tc_lattice(
        emb, gw_pad, valfw, valbw,
        fw_Wh.astype(bf), fw_Wwx.astype(bf), fw_Wwh.astype(bf),
        fw_bw.reshape(1, 3 * H), fw_Wlc.astype(bf),
        bw_Wh.astype(bf), bw_Wwx.astype(bf), bw_Wwh.astype(bf),
        bw_bw.reshape(1, 3 * H), bw_Wlc.astype(bf),
        fw_Wx, fw_b.reshape(1, 4 * H), fw_Wlx, fw_bl.reshape(1, H),
        bw_Wx, bw_b.reshape(1, 4 * H), bw_Wlx, bw_bl.reshape(1, H))
    return hs[None, :, :]
